# Initial kernel scaffold; baseline (speedup 1.0000x reference)
import jax
import jax.numpy as jnp
from jax.experimental import pallas as pl


def kernel(xyz, features, xyz_surface, features_surface, W_fc1, b_fc1, W_fc1s, b_fc1s, W_fc2, b_fc2, W_d1, b_d1, W_d2, b_d2, W_g1, b_g1, W_g2, b_g2, W_k, W_v):
    raise NotImplementedError("write your pallas kernel here")


# speedup vs baseline: 13.6998x; 13.6998x over previous
import functools
import math

import jax
import jax.numpy as jnp
from jax import lax
from jax.experimental import pallas as pl
from jax.experimental.pallas import tpu as pltpu
from jax.experimental.pallas import tpu_sc as plsc


def _prep_body(fs_ref, xs_ref, wf_ref, bf_ref, wk_ref, wv_ref, wg1_ref,
               wd1_ref, tbl_ref):
    x = jnp.dot(fs_ref[0], wf_ref[...], preferred_element_type=jnp.float32)
    x = x + bf_ref[...]
    wkg1 = jnp.dot(wk_ref[...], wg1_ref[...], preferred_element_type=jnp.float32)
    tbl_ref[0, :, 0:128] = jnp.dot(x, wkg1, preferred_element_type=jnp.float32)
    tbl_ref[0, :, 128:256] = jnp.dot(x, wv_ref[...], preferred_element_type=jnp.float32)
    tbl_ref[0, :, 256:384] = jnp.dot(xs_ref[0], wd1_ref[...],
                                     preferred_element_type=jnp.float32)


def _prep_table(features_surface, xyz_s_pad, W_fc1s, b_fc1s, W_k, W_v, W_g1,
                W_d1p):
    B, N2, D = features_surface.shape
    return pl.pallas_call(
        _prep_body,
        grid=(B,),
        in_specs=[
            pl.BlockSpec((1, N2, D), lambda b: (b, 0, 0)),
            pl.BlockSpec((1, N2, 16), lambda b: (b, 0, 0)),
            pl.BlockSpec((D, 128), lambda b: (0, 0)),
            pl.BlockSpec((1, 128), lambda b: (0, 0)),
            pl.BlockSpec((128, 128), lambda b: (0, 0)),
            pl.BlockSpec((128, 128), lambda b: (0, 0)),
            pl.BlockSpec((128, 128), lambda b: (0, 0)),
            pl.BlockSpec((16, 128), lambda b: (0, 0)),
        ],
        out_specs=pl.BlockSpec((1, N2, 384), lambda b: (b, 0, 0)),
        out_shape=jax.ShapeDtypeStruct((B, N2, 384), jnp.float32),
    )(features_surface, xyz_s_pad, W_fc1s, b_fc1s.reshape(1, 128), W_k, W_v,
      W_g1, W_d1p)


def _topk_body(K, N2, xq_ref, ysT_ref, idx_ref):
    b = pl.program_id(0)
    xq = xq_ref[0]
    BQ = xq.shape[0]
    d0 = xq[:, 0:1] - ysT_ref[0, 0:1, :]
    d1 = xq[:, 1:2] - ysT_ref[0, 1:2, :]
    d2 = xq[:, 2:3] - ysT_ref[0, 2:3, :]
    dist = (d0 * d0 + d1 * d1) + d2 * d2
    iota = lax.broadcasted_iota(jnp.int32, (BQ, N2), 1)
    for k in range(K):
        m = jnp.min(dist, axis=1, keepdims=True)
        amin = jnp.min(jnp.where(dist == m, iota, N2), axis=1,
                       keepdims=True)
        idx_ref[0, k, :] = amin[:, 0] + b * N2
        dist = jnp.where(iota == amin, jnp.inf, dist)


def _topk(xyz_q_pad, xyz_sT_pad, K, BQ=256):
    B, N1, _ = xyz_q_pad.shape
    N2 = xyz_sT_pad.shape[2]
    return pl.pallas_call(
        functools.partial(_topk_body, K, N2),
        grid=(B, N1 // BQ),
        in_specs=[
            pl.BlockSpec((1, BQ, 16), lambda b, i: (b, i, 0)),
            pl.BlockSpec((1, 16, N2), lambda b, i: (b, 0, 0)),
        ],
        out_specs=pl.BlockSpec((1, K, BQ), lambda b, i: (b, 0, i)),
        out_shape=jax.ShapeDtypeStruct((B, K, N1), jnp.int32),
        compiler_params=pltpu.CompilerParams(
            dimension_semantics=("parallel", "parallel")),
    )(xyz_q_pad, xyz_sT_pad)


def _sc_gather(tbl_flat, idx_mat):
    NCHUNK, CW = idx_mat.shape
    W = tbl_flat.shape[1]
    TOT = NCHUNK * CW
    info = plsc.get_sparse_core_info()
    NW = info.num_cores * info.num_subcores
    per_w = NCHUNK // NW

    mesh = plsc.VectorSubcoreMesh(core_axis_name="c", subcore_axis_name="s")

    @functools.partial(
        pl.kernel,
        mesh=mesh,
        out_type=jax.ShapeDtypeStruct((TOT, W), jnp.float32),
        scratch_types=[
            pltpu.VMEM((per_w, CW), jnp.int32),
            pltpu.VMEM((CW, W), jnp.float32),
            pltpu.VMEM((CW, W), jnp.float32),
            pltpu.SemaphoreType.DMA,
            pltpu.SemaphoreType.DMA,
        ],
    )
    def gather_k(tbl_hbm, idx_hbm, out_hbm, idx_v, buf0, buf1, sem0, sem1):
        wid = lax.axis_index("s") * info.num_cores + lax.axis_index("c")
        row0 = wid * per_w
        pltpu.sync_copy(idx_hbm.at[pl.ds(row0, per_w)], idx_v)
        bufs = (buf0, buf1)
        sems = (sem0, sem1)
        cps = [None, None]
        for j in range(per_w):
            p = j % 2
            if cps[p] is not None:
                cps[p].wait()
                pltpu.sync_copy(bufs[p],
                                out_hbm.at[pl.ds((row0 + j - 2) * CW, CW)])
            cps[p] = pltpu.async_copy(tbl_hbm.at[idx_v.at[j]], bufs[p], sems[p])
        for j in range(per_w - 2, per_w):
            p = j % 2
            cps[p].wait()
            pltpu.sync_copy(bufs[p], out_hbm.at[pl.ds((row0 + j) * CW, CW)])

    return gather_k(tbl_flat, idx_mat)


def _attn_body(K, f_ref, xq_ref, tg_ref,
               wq_ref, bq_ref, wd1_ref, bd1_ref, wd2_ref, bd2_ref,
               wg1_ref, bg1_ref, wg2_ref, bg2_ref, wf2_ref, bf2_ref,
               res_ref, attn_ref, h_scr, lg_scr):
    pre = f_ref[0]
    wfcg1 = jnp.dot(wq_ref[...], wg1_ref[...], preferred_element_type=jnp.float32)
    wdg = jnp.dot(wd2_ref[...], wg1_ref[...], preferred_element_type=jnp.float32)
    bias1 = jnp.dot(bq_ref[...] + bd2_ref[...], wg1_ref[...],
                    preferred_element_type=jnp.float32) + bg1_ref[...]
    q1 = jnp.dot(pre, wfcg1, preferred_element_type=jnp.float32) + bias1
    sq = jnp.dot(xq_ref[0], wd1_ref[...],
                 preferred_element_type=jnp.float32) + bd1_ref[...]
    scale = 1.0 / math.sqrt(128.0)
    m = jnp.full(pre.shape, -jnp.inf, dtype=jnp.float32)
    for k in range(K):
        h = jnp.maximum(sq - tg_ref[0, k, :, 256:384], 0.0)
        h_scr[k] = h
        hh = jnp.maximum(
            q1 - tg_ref[0, k, :, 0:128]
            + jnp.dot(h, wdg, preferred_element_type=jnp.float32), 0.0)
        lg = (jnp.dot(hh, wg2_ref[...], preferred_element_type=jnp.float32)
              + bg2_ref[...]) * scale
        lg_scr[k] = lg
        m = jnp.maximum(m, lg)
    s = jnp.zeros(pre.shape, jnp.float32)
    r = jnp.zeros(pre.shape, jnp.float32)
    for k in range(K):
        e = jnp.exp(lg_scr[k] - m)
        s = s + e
        pe = jnp.dot(h_scr[k], wd2_ref[...],
                     preferred_element_type=jnp.float32) + bd2_ref[...]
        r = r + e * (tg_ref[0, k, :, 128:256] + pe)
        attn_ref[0, :, k * 128:(k + 1) * 128] = e
    inv = 1.0 / s
    for k in range(K):
        attn_ref[0, :, k * 128:(k + 1) * 128] = (
            attn_ref[0, :, k * 128:(k + 1) * 128] * inv)
    res_ref[0] = pre + (
        jnp.dot(r * inv, wf2_ref[...], preferred_element_type=jnp.float32)
        + bf2_ref[...])


def _attention(features, xyz_q_pad, tg, K,
               W_fc1, b_fc1, W_d1p, b_d1, W_d2, b_d2,
               W_g1, b_g1, W_g2, b_g2, W_fc2, b_fc2, BQ=256):
    B, N1, D = features.shape
    wspec = lambda shp: pl.BlockSpec(shp, lambda b, i: tuple(0 for _ in shp))
    return pl.pallas_call(
        functools.partial(_attn_body, K),
        grid=(B, N1 // BQ),
        in_specs=[
            pl.BlockSpec((1, BQ, D), lambda b, i: (b, i, 0)),
            pl.BlockSpec((1, BQ, 16), lambda b, i: (b, i, 0)),
            pl.BlockSpec((1, K, BQ, 384), lambda b, i: (b, 0, i, 0)),
            wspec((D, 128)), wspec((1, 128)),
            wspec((16, 128)), wspec((1, 128)),
            wspec((128, 128)), wspec((1, 128)),
            wspec((128, 128)), wspec((1, 128)),
            wspec((128, 128)), wspec((1, 128)),
            wspec((128, D)), wspec((1, D)),
        ],
        out_specs=[
            pl.BlockSpec((1, BQ, D), lambda b, i: (b, i, 0)),
            pl.BlockSpec((1, BQ, K * 128), lambda b, i: (b, i, 0)),
        ],
        out_shape=[
            jax.ShapeDtypeStruct((B, N1, D), jnp.float32),
            jax.ShapeDtypeStruct((B, N1, K * 128), jnp.float32),
        ],
        scratch_shapes=[
            pltpu.VMEM((K, BQ, 128), jnp.float32),
            pltpu.VMEM((K, BQ, 128), jnp.float32),
        ],
        compiler_params=pltpu.CompilerParams(
            dimension_semantics=("parallel", "parallel")),
    )(features, xyz_q_pad, tg,
      W_fc1, b_fc1.reshape(1, 128), W_d1p, b_d1.reshape(1, 128),
      W_d2, b_d2.reshape(1, 128), W_g1, b_g1.reshape(1, 128),
      W_g2, b_g2.reshape(1, 128), W_fc2, b_fc2.reshape(1, D))


def kernel(xyz, features, xyz_surface, features_surface,
           W_fc1, b_fc1, W_fc1s, b_fc1s, W_fc2, b_fc2,
           W_d1, b_d1, W_d2, b_d2, W_g1, b_g1, W_g2, b_g2,
           W_k, W_v):
    B, N1, _ = xyz.shape
    N2 = xyz_surface.shape[1]
    K = 16
    D = features.shape[-1]

    pad_q = jnp.zeros((B, N1, 13), jnp.float32)
    pad_s = jnp.zeros((B, N2, 13), jnp.float32)
    xyz_q_pad = jnp.concatenate([xyz, pad_q], axis=-1)
    xyz_s_pad = jnp.concatenate([xyz_surface, pad_s], axis=-1)
    xyz_sT_pad = jnp.transpose(xyz_s_pad, (0, 2, 1))
    W_d1p = jnp.concatenate([W_d1, jnp.zeros((13, 128), jnp.float32)], axis=0)

    tbl = _prep_table(features_surface, xyz_s_pad, W_fc1s, b_fc1s, W_k, W_v,
                      W_g1, W_d1p)
    gidx = _topk(xyz_q_pad, xyz_sT_pad, K)

    CW = 128
    idx_mat = gidx.reshape(B * K * N1 // CW, CW)
    tg_flat = _sc_gather(tbl.reshape(B * N2, 384), idx_mat)
    tg = tg_flat.reshape(B, K, N1, 384)

    res, attn = _attention(features, xyz_q_pad, tg, K,
                           W_fc1, b_fc1, W_d1p, b_d1, W_d2, b_d2,
                           W_g1, b_g1, W_g2, b_g2, W_fc2, b_fc2)
    return res, attn.reshape(B, N1, K, 128)


# speedup vs baseline: 15.5425x; 1.1345x over previous
import functools
import math

import jax
import jax.numpy as jnp
from jax import lax
from jax.experimental import pallas as pl
from jax.experimental.pallas import tpu as pltpu
from jax.experimental.pallas import tpu_sc as plsc


def _prep_body(fs_ref, xs_ref, wf_ref, bf_ref, wk_ref, wv_ref, wg1_ref,
               wd1_ref, tbl_ref):
    x = jnp.dot(fs_ref[0], wf_ref[...], preferred_element_type=jnp.float32)
    x = x + bf_ref[...]
    wkg1 = jnp.dot(wk_ref[...], wg1_ref[...], preferred_element_type=jnp.float32)
    tbl_ref[0, :, 0:128] = jnp.dot(x, wkg1, preferred_element_type=jnp.float32)
    tbl_ref[0, :, 128:256] = jnp.dot(x, wv_ref[...], preferred_element_type=jnp.float32)
    tbl_ref[0, :, 256:384] = jnp.dot(xs_ref[0], wd1_ref[...],
                                     preferred_element_type=jnp.float32)


def _prep_table(features_surface, xyz_s_pad, W_fc1s, b_fc1s, W_k, W_v, W_g1,
                W_d1p):
    B, N2, D = features_surface.shape
    return pl.pallas_call(
        _prep_body,
        grid=(B,),
        in_specs=[
            pl.BlockSpec((1, N2, D), lambda b: (b, 0, 0)),
            pl.BlockSpec((1, N2, 16), lambda b: (b, 0, 0)),
            pl.BlockSpec((D, 128), lambda b: (0, 0)),
            pl.BlockSpec((1, 128), lambda b: (0, 0)),
            pl.BlockSpec((128, 128), lambda b: (0, 0)),
            pl.BlockSpec((128, 128), lambda b: (0, 0)),
            pl.BlockSpec((128, 128), lambda b: (0, 0)),
            pl.BlockSpec((16, 128), lambda b: (0, 0)),
        ],
        out_specs=pl.BlockSpec((1, N2, 384), lambda b: (b, 0, 0)),
        out_shape=jax.ShapeDtypeStruct((B, N2, 384), jnp.float32),
    )(features_surface, xyz_s_pad, W_fc1s, b_fc1s.reshape(1, 128), W_k, W_v,
      W_g1, W_d1p)


def _topk_body(K, N2, xq_ref, ysT_ref, idx_ref):
    b = pl.program_id(0)
    xq = xq_ref[0]
    BQ = xq.shape[0]
    d0 = xq[:, 0:1] - ysT_ref[0, 0:1, :]
    d1 = xq[:, 1:2] - ysT_ref[0, 1:2, :]
    d2 = xq[:, 2:3] - ysT_ref[0, 2:3, :]
    dist = (d0 * d0 + d1 * d1) + d2 * d2
    iota = lax.broadcasted_iota(jnp.int32, (BQ, N2), 1)
    for k in range(K):
        m = jnp.min(dist, axis=1, keepdims=True)
        amin = jnp.min(jnp.where(dist == m, iota, N2), axis=1,
                       keepdims=True)
        idx_ref[0, k, :] = amin[:, 0] + b * N2
        if k + 1 < K:
            dist = jnp.where(iota == amin, jnp.inf, dist)


def _topk_seg(xyz_q_pad, xyz_sT_pad, K, seg0, SEG, BQ=256):
    B, N1, _ = xyz_q_pad.shape
    N2 = xyz_sT_pad.shape[2]
    nb = SEG // BQ
    return pl.pallas_call(
        functools.partial(_topk_body, K, N2),
        grid=(B, nb),
        in_specs=[
            pl.BlockSpec((1, BQ, 16),
                         lambda b, i: (b, seg0 * nb + i, 0)),
            pl.BlockSpec((1, 16, N2), lambda b, i: (b, 0, 0)),
        ],
        out_specs=pl.BlockSpec((1, K, BQ), lambda b, i: (b, 0, i)),
        out_shape=jax.ShapeDtypeStruct((B, K, SEG), jnp.int32),
        compiler_params=pltpu.CompilerParams(
            dimension_semantics=("parallel", "parallel")),
    )(xyz_q_pad, xyz_sT_pad)


def _sc_gather(tbl_flat, idx_mat):
    NCHUNK, CW = idx_mat.shape
    W = tbl_flat.shape[1]
    TOT = NCHUNK * CW
    info = plsc.get_sparse_core_info()
    NW = info.num_cores * info.num_subcores
    per_w = NCHUNK // NW

    mesh = plsc.VectorSubcoreMesh(core_axis_name="c", subcore_axis_name="s")

    @functools.partial(
        pl.kernel,
        mesh=mesh,
        out_type=jax.ShapeDtypeStruct((TOT, W), jnp.float32),
        scratch_types=[
            pltpu.VMEM((per_w, CW), jnp.int32),
            pltpu.VMEM((CW, W), jnp.float32),
            pltpu.VMEM((CW, W), jnp.float32),
            pltpu.SemaphoreType.DMA,
            pltpu.SemaphoreType.DMA,
        ],
    )
    def gather_k(tbl_hbm, idx_hbm, out_hbm, idx_v, buf0, buf1, sem0, sem1):
        wid = lax.axis_index("s") * info.num_cores + lax.axis_index("c")
        row0 = wid * per_w
        pltpu.sync_copy(idx_hbm.at[pl.ds(row0, per_w)], idx_v)
        bufs = (buf0, buf1)
        sems = (sem0, sem1)
        cps = [None, None]
        for j in range(per_w):
            p = j % 2
            if cps[p] is not None:
                cps[p].wait()
                pltpu.sync_copy(bufs[p],
                                out_hbm.at[pl.ds((row0 + j - 2) * CW, CW)])
            cps[p] = pltpu.async_copy(tbl_hbm.at[idx_v.at[j]], bufs[p], sems[p])
        for j in range(per_w - 2, per_w):
            p = j % 2
            cps[p].wait()
            pltpu.sync_copy(bufs[p], out_hbm.at[pl.ds((row0 + j) * CW, CW)])

    return gather_k(tbl_flat, idx_mat)


def _attn_body(K, f_ref, xq_ref, tg_ref,
               wq_ref, bq_ref, wd1_ref, bd1_ref, wd2_ref, bd2_ref,
               wg1_ref, bg1_ref, wg2_ref, bg2_ref, wf2_ref, bf2_ref,
               res_in_ref, attn_in_ref,
               res_ref, attn_ref, h_scr, lg_scr):
    del res_in_ref, attn_in_ref
    pre = f_ref[0]
    wfcg1 = jnp.dot(wq_ref[...], wg1_ref[...], preferred_element_type=jnp.float32)
    wdg = jnp.dot(wd2_ref[...], wg1_ref[...], preferred_element_type=jnp.float32)
    bias1 = jnp.dot(bq_ref[...] + bd2_ref[...], wg1_ref[...],
                    preferred_element_type=jnp.float32) + bg1_ref[...]
    q1 = jnp.dot(pre, wfcg1, preferred_element_type=jnp.float32) + bias1
    sq = jnp.dot(xq_ref[0], wd1_ref[...],
                 preferred_element_type=jnp.float32) + bd1_ref[...]
    scale = 1.0 / math.sqrt(128.0)
    m = jnp.full(pre.shape, -jnp.inf, dtype=jnp.float32)
    for k in range(K):
        h = jnp.maximum(sq - tg_ref[0, k, :, 256:384], 0.0)
        h_scr[k] = h
        hh = jnp.maximum(
            q1 - tg_ref[0, k, :, 0:128]
            + jnp.dot(h, wdg, preferred_element_type=jnp.float32), 0.0)
        lg = (jnp.dot(hh, wg2_ref[...], preferred_element_type=jnp.float32)
              + bg2_ref[...]) * scale
        lg_scr[k] = lg
        m = jnp.maximum(m, lg)
    s = jnp.zeros(pre.shape, jnp.float32)
    r = jnp.zeros(pre.shape, jnp.float32)
    for k in range(K):
        e = jnp.exp(lg_scr[k] - m)
        s = s + e
        pe = jnp.dot(h_scr[k], wd2_ref[...],
                     preferred_element_type=jnp.float32) + bd2_ref[...]
        r = r + e * (tg_ref[0, k, :, 128:256] + pe)
        attn_ref[0, :, k * 128:(k + 1) * 128] = e
    inv = 1.0 / s
    for k in range(K):
        attn_ref[0, :, k * 128:(k + 1) * 128] = (
            attn_ref[0, :, k * 128:(k + 1) * 128] * inv)
    res_ref[0] = pre + (
        jnp.dot(r * inv, wf2_ref[...], preferred_element_type=jnp.float32)
        + bf2_ref[...])


def _attention_seg(features, xyz_q_pad, tg, K, seg0, SEG, weights,
                   res_prev, attn_prev, BQ=256):
    B, N1, D = features.shape
    nb = SEG // BQ
    (W_fc1, b_fc1, W_d1p, b_d1, W_d2, b_d2,
     W_g1, b_g1, W_g2, b_g2, W_fc2, b_fc2) = weights
    wspec = lambda shp: pl.BlockSpec(shp, lambda b, i: tuple(0 for _ in shp))
    anyspec = pl.BlockSpec(memory_space=pl.ANY)
    has_prev = res_prev is not None
    extra_in = (res_prev, attn_prev) if has_prev else ()
    body = functools.partial(_attn_body, K) if has_prev else (
        lambda *refs: _attn_body(K, *refs[:15], None, None, *refs[15:]))
    return pl.pallas_call(
        body,
        grid=(B, nb),
        in_specs=[
            pl.BlockSpec((1, BQ, D), lambda b, i: (b, seg0 * nb + i, 0)),
            pl.BlockSpec((1, BQ, 16), lambda b, i: (b, seg0 * nb + i, 0)),
            pl.BlockSpec((1, K, BQ, 384), lambda b, i: (b, 0, i, 0)),
            wspec((D, 128)), wspec((1, 128)),
            wspec((16, 128)), wspec((1, 128)),
            wspec((128, 128)), wspec((1, 128)),
            wspec((128, 128)), wspec((1, 128)),
            wspec((128, 128)), wspec((1, 128)),
            wspec((128, D)), wspec((1, D)),
        ] + ([anyspec, anyspec] if has_prev else []),
        out_specs=[
            pl.BlockSpec((1, BQ, D), lambda b, i: (b, seg0 * nb + i, 0)),
            pl.BlockSpec((1, BQ, K * 128), lambda b, i: (b, seg0 * nb + i, 0)),
        ],
        out_shape=[
            jax.ShapeDtypeStruct((B, N1, D), jnp.float32),
            jax.ShapeDtypeStruct((B, N1, K * 128), jnp.float32),
        ],
        scratch_shapes=[
            pltpu.VMEM((K, BQ, 128), jnp.float32),
            pltpu.VMEM((K, BQ, 128), jnp.float32),
        ],
        input_output_aliases={15: 0, 16: 1} if has_prev else {},
        compiler_params=pltpu.CompilerParams(
            dimension_semantics=("parallel", "parallel")),
    )(features, xyz_q_pad, tg,
      W_fc1, b_fc1.reshape(1, 128), W_d1p, b_d1.reshape(1, 128),
      W_d2, b_d2.reshape(1, 128), W_g1, b_g1.reshape(1, 128),
      W_g2, b_g2.reshape(1, 128), W_fc2, b_fc2.reshape(1, D),
      *extra_in)


def kernel(xyz, features, xyz_surface, features_surface,
           W_fc1, b_fc1, W_fc1s, b_fc1s, W_fc2, b_fc2,
           W_d1, b_d1, W_d2, b_d2, W_g1, b_g1, W_g2, b_g2,
           W_k, W_v):
    B, N1, _ = xyz.shape
    N2 = xyz_surface.shape[1]
    K = 16
    D = features.shape[-1]
    NSEG = 4
    SEG = N1 // NSEG
    CW = 128

    pad_q = jnp.zeros((B, N1, 13), jnp.float32)
    pad_s = jnp.zeros((B, N2, 13), jnp.float32)
    xyz_q_pad = jnp.concatenate([xyz, pad_q], axis=-1)
    xyz_s_pad = jnp.concatenate([xyz_surface, pad_s], axis=-1)
    xyz_sT_pad = jnp.transpose(xyz_s_pad, (0, 2, 1))
    W_d1p = jnp.concatenate([W_d1, jnp.zeros((13, 128), jnp.float32)], axis=0)

    tbl = _prep_table(features_surface, xyz_s_pad, W_fc1s, b_fc1s, W_k, W_v,
                      W_g1, W_d1p)
    tbl_flat = tbl.reshape(B * N2, 384)
    weights = (W_fc1, b_fc1, W_d1p, b_d1, W_d2, b_d2,
               W_g1, b_g1, W_g2, b_g2, W_fc2, b_fc2)

    gidx = [_topk_seg(xyz_q_pad, xyz_sT_pad, K, h, SEG) for h in range(NSEG)]
    tgs = [_sc_gather(tbl_flat, gidx[h].reshape(B * K * SEG // CW, CW))
           .reshape(B, K, SEG, 384) for h in range(NSEG)]

    res, attn = None, None
    for h in range(NSEG):
        res, attn = _attention_seg(features, xyz_q_pad, tgs[h], K, h, SEG,
                                   weights, res, attn)
    return res, attn.reshape(B, N1, K, 128)
